# baseline (speedup 1.0000x reference)
import functools

import jax
import jax.numpy as jnp
from jax import lax
from jax.experimental import pallas as pl
from jax.experimental.pallas import tpu as pltpu
from jax.experimental.pallas import tpu_sc as plsc

RANK = 32
EMBED_DIM = 128

_NC = 2
_NS = 16
_NW = _NC * _NS

ROWS_PER_GATHER = 128
GATHERS_PER_STEP = 8
NCHUNKS = 5
MM_BLK = 2048


def _sc_gather(idx2d, table, n_rows128):
    steps_per_worker = n_rows128 // (_NW * GATHERS_PER_STEP)
    n_steps = n_rows128 // GATHERS_PER_STEP
    mesh = plsc.VectorSubcoreMesh(core_axis_name="c", subcore_axis_name="s")

    @functools.partial(
        pl.kernel,
        mesh=mesh,
        out_type=jax.ShapeDtypeStruct(
            (n_steps, GATHERS_PER_STEP, ROWS_PER_GATHER, RANK), jnp.bfloat16),
        scratch_types=[
            pltpu.VMEM((GATHERS_PER_STEP, ROWS_PER_GATHER), jnp.int32),
            pltpu.VMEM((GATHERS_PER_STEP, ROWS_PER_GATHER, RANK), jnp.bfloat16),
            pltpu.SemaphoreType.DMA,
        ],
        compiler_params=pltpu.CompilerParams(use_tc_tiling_on_sc=False),
    )
    def gather_kernel(idx_hbm, table_hbm, h_hbm, idx_v, rows_v, sem):
        wid = lax.axis_index("s") * _NC + lax.axis_index("c")
        step0 = wid * steps_per_worker

        def step(g, carry):
            s = step0 + g
            pltpu.sync_copy(
                idx_hbm.at[pl.ds(s * GATHERS_PER_STEP, GATHERS_PER_STEP)],
                idx_v)
            copies = []
            for j in range(GATHERS_PER_STEP):
                copies.append(
                    pltpu.async_copy(table_hbm.at[idx_v.at[j]], rows_v.at[j],
                                     sem))
            for c in copies:
                c.wait()
            pltpu.sync_copy(rows_v, h_hbm.at[s])
            return carry

        lax.fori_loop(0, steps_per_worker, step, 0)

    return gather_kernel(idx2d, table)


def _tc_pack_table(lora_a):
    v = lora_a.shape[0]
    at = lora_a.T
    blk_cols = 8192
    grid = -(-v // blk_cols)
    out_rows = grid * (blk_cols // 4)

    def pack_body(a_ref, o_ref):
        xb = a_ref[...].astype(jnp.bfloat16)
        xt = xb.T
        x3 = xt.reshape(blk_cols // 4, 4, RANK)
        o_ref[...] = jnp.concatenate([x3[:, k, :] for k in range(4)], axis=1)

    pack = pl.pallas_call(
        pack_body,
        grid=(grid,),
        in_specs=[pl.BlockSpec((RANK, blk_cols), lambda i: (0, i))],
        out_specs=pl.BlockSpec((blk_cols // 4, 128), lambda i: (i, 0)),
        out_shape=jax.ShapeDtypeStruct((out_rows, 128), jnp.bfloat16),
    )(at)
    return pack.reshape(out_rows * 4, RANK)


def _tc_matmul_chunk(h2c, wcat, out_prev, c, q_rows, n):
    nb = q_rows // MM_BLK
    blk0 = c * nb

    def mm_body(h_ref, w_ref, o_ref):
        h = h_ref[...].astype(jnp.float32)
        cat = jax.lax.dot_general(
            h, w_ref[...], (((1,), (0,)), ((), ())),
            preferred_element_type=jnp.float32)
        o_ref[...] = cat.reshape(4 * MM_BLK, EMBED_DIM)

    in_specs = [
        pl.BlockSpec((MM_BLK, 128), lambda j: (j, 0)),
        pl.BlockSpec((128, 512), lambda j: (0, 0)),
    ]
    out_spec = pl.BlockSpec((4 * MM_BLK, EMBED_DIM), lambda j: (blk0 + j, 0))
    out_sds = jax.ShapeDtypeStruct((n, EMBED_DIM), jnp.float32)

    if out_prev is None:
        return pl.pallas_call(
            mm_body,
            grid=(nb,),
            in_specs=in_specs,
            out_specs=out_spec,
            out_shape=out_sds,
        )(h2c, wcat)

    def mm_body_acc(h_ref, w_ref, prev_ref, o_ref):
        del prev_ref
        mm_body(h_ref, w_ref, o_ref)

    return pl.pallas_call(
        mm_body_acc,
        grid=(nb,),
        in_specs=in_specs + [pl.BlockSpec(memory_space=pl.ANY)],
        out_specs=out_spec,
        out_shape=out_sds,
        input_output_aliases={2: 0},
    )(h2c, wcat, out_prev)


def kernel(x, lora_a, lora_b):
    batch, seq = x.shape
    n = batch * seq
    chunk_rows = n // NCHUNKS
    q_rows = chunk_rows // 4
    nr128 = chunk_rows // ROWS_PER_GATHER
    x_flat = x.reshape(n).astype(jnp.int32)
    table_lin = _tc_pack_table(lora_a)
    b_t = lora_b.T.astype(jnp.float32)
    wcat = jnp.zeros((128, 4 * EMBED_DIM), jnp.float32)
    for k in range(4):
        wcat = wcat.at[k * RANK:(k + 1) * RANK,
                       k * EMBED_DIM:(k + 1) * EMBED_DIM].set(b_t)

    out = None
    for c in range(NCHUNKS):
        xc = lax.dynamic_slice(x_flat, (c * chunk_rows,), (chunk_rows,))
        idx2d = xc.reshape(nr128, ROWS_PER_GATHER)
        h4 = _sc_gather(idx2d, table_lin, nr128)
        h2c = h4.reshape(q_rows, 128)
        out = _tc_matmul_chunk(h2c, wcat, out, c, q_rows, n)
    return out.reshape(batch, seq, EMBED_DIM)


# speedup vs baseline: 1.6045x; 1.6045x over previous
import functools

import jax
import jax.numpy as jnp
from jax import lax
from jax.experimental import pallas as pl
from jax.experimental.pallas import tpu as pltpu
from jax.experimental.pallas import tpu_sc as plsc

RANK = 32
EMBED_DIM = 128

_NC = 2
_NS = 16
_NW = _NC * _NS

ROWS_PER_GATHER = 128
GATHERS_PER_STEP = 8
CHUNK_STEPS = (32, 96, 160, 224, 288)
MM_BLK = 2048


def _sc_gather(idx2d, table, n_rows128):
    steps_per_worker = n_rows128 // (_NW * GATHERS_PER_STEP)
    n_steps = n_rows128 // GATHERS_PER_STEP
    mesh = plsc.VectorSubcoreMesh(core_axis_name="c", subcore_axis_name="s")

    @functools.partial(
        pl.kernel,
        mesh=mesh,
        out_type=jax.ShapeDtypeStruct(
            (n_steps, GATHERS_PER_STEP, ROWS_PER_GATHER, RANK), jnp.float32),
        scratch_types=[
            pltpu.VMEM((GATHERS_PER_STEP, ROWS_PER_GATHER), jnp.int32),
            pltpu.VMEM((GATHERS_PER_STEP, ROWS_PER_GATHER, RANK), jnp.float32),
            pltpu.SemaphoreType.DMA,
        ],
        compiler_params=pltpu.CompilerParams(use_tc_tiling_on_sc=False),
    )
    def gather_kernel(idx_hbm, table_hbm, h_hbm, idx_v, rows_v, sem):
        wid = lax.axis_index("s") * _NC + lax.axis_index("c")
        step0 = wid * steps_per_worker

        def step(g, carry):
            s = step0 + g
            pltpu.sync_copy(
                idx_hbm.at[pl.ds(s * GATHERS_PER_STEP, GATHERS_PER_STEP)],
                idx_v)
            copies = []
            for j in range(GATHERS_PER_STEP):
                copies.append(
                    pltpu.async_copy(table_hbm.at[idx_v.at[j]], rows_v.at[j],
                                     sem))
            for c in copies:
                c.wait()
            pltpu.sync_copy(rows_v, h_hbm.at[s])
            return carry

        lax.fori_loop(0, steps_per_worker, step, 0)

    return gather_kernel(idx2d, table)


def _tc_pack_table(lora_a):
    v = lora_a.shape[0]
    at = lora_a.T
    blk_cols = 8192
    grid = -(-v // blk_cols)
    out_rows = grid * (blk_cols // 4)

    def pack_body(a_ref, o_ref):
        xt = a_ref[...].T
        x3 = xt.reshape(blk_cols // 4, 4, RANK)
        o_ref[...] = jnp.concatenate([x3[:, k, :] for k in range(4)], axis=1)

    pack = pl.pallas_call(
        pack_body,
        grid=(grid,),
        in_specs=[pl.BlockSpec((RANK, blk_cols), lambda i: (0, i))],
        out_specs=pl.BlockSpec((blk_cols // 4, 128), lambda i: (i, 0)),
        out_shape=jax.ShapeDtypeStruct((out_rows, 128), jnp.float32),
    )(at)
    return pack.reshape(out_rows * 4, RANK)


def _tc_matmul_chunk(h2c, wcat, out_prev, blk0, q_rows, n):
    nb = q_rows // MM_BLK

    def mm_body(h_ref, w_ref, o_ref):
        cat = jax.lax.dot_general(
            h_ref[...], w_ref[...], (((1,), (0,)), ((), ())),
            preferred_element_type=jnp.float32)
        o_ref[...] = cat.reshape(4 * MM_BLK, EMBED_DIM)

    in_specs = [
        pl.BlockSpec((MM_BLK, 128), lambda j: (j, 0)),
        pl.BlockSpec((128, 512), lambda j: (0, 0)),
    ]
    out_spec = pl.BlockSpec((4 * MM_BLK, EMBED_DIM), lambda j: (blk0 + j, 0))
    out_sds = jax.ShapeDtypeStruct((n, EMBED_DIM), jnp.float32)

    if out_prev is None:
        return pl.pallas_call(
            mm_body,
            grid=(nb,),
            in_specs=in_specs,
            out_specs=out_spec,
            out_shape=out_sds,
        )(h2c, wcat)

    def mm_body_acc(h_ref, w_ref, prev_ref, o_ref):
        del prev_ref
        mm_body(h_ref, w_ref, o_ref)

    return pl.pallas_call(
        mm_body_acc,
        grid=(nb,),
        in_specs=in_specs + [pl.BlockSpec(memory_space=pl.ANY)],
        out_specs=out_spec,
        out_shape=out_sds,
        input_output_aliases={2: 0},
    )(h2c, wcat, out_prev)


def kernel(x, lora_a, lora_b):
    batch, seq = x.shape
    n = batch * seq
    x_flat = x.reshape(n).astype(jnp.int32)
    table_lin = _tc_pack_table(lora_a)
    b_t = lora_b.T.astype(jnp.float32)
    wcat = jnp.zeros((128, 4 * EMBED_DIM), jnp.float32)
    for k in range(4):
        wcat = wcat.at[k * RANK:(k + 1) * RANK,
                       k * EMBED_DIM:(k + 1) * EMBED_DIM].set(b_t)

    out = None
    row0 = 0
    for steps in CHUNK_STEPS:
        chunk_rows = steps * GATHERS_PER_STEP * ROWS_PER_GATHER
        q_rows = chunk_rows // 4
        nr128 = chunk_rows // ROWS_PER_GATHER
        xc = lax.dynamic_slice(x_flat, (row0,), (chunk_rows,))
        idx2d = xc.reshape(nr128, ROWS_PER_GATHER)
        h4 = _sc_gather(idx2d, table_lin, nr128)
        h2c = h4.reshape(q_rows, 128)
        out = _tc_matmul_chunk(h2c, wcat, out, row0 // (4 * MM_BLK), q_rows, n)
        row0 += chunk_rows
    return out.reshape(batch, seq, EMBED_DIM)
